# Initial kernel scaffold; baseline (speedup 1.0000x reference)
import jax
import jax.numpy as jnp
from jax.experimental import pallas as pl


def kernel(points1, points2, colors1):
    raise NotImplementedError("write your pallas kernel here")


# speedup vs baseline: 29.3427x; 29.3427x over previous
import functools

import jax
import jax.numpy as jnp
from jax.experimental import pallas as pl
from jax.experimental.pallas import tpu as pltpu

_R = 256


def _knn_body(p2_ref, p1t_ref, caug_ref, out_ref):
    q = p2_ref[0]
    k = p1t_ref[0]
    dx = q[:, 0:1] - k[0:1, :]
    dy = q[:, 1:2] - k[1:2, :]
    dz = q[:, 2:3] - k[2:3, :]
    dist = (dx * dx + dy * dy) + dz * dz
    big = jnp.float32(jnp.inf)
    m1 = jnp.min(dist, axis=1, keepdims=True)
    d2 = jnp.where(dist == m1, big, dist)
    m2 = jnp.min(d2, axis=1, keepdims=True)
    d3 = jnp.where(d2 == m2, big, d2)
    m3 = jnp.min(d3, axis=1, keepdims=True)
    maskf = (dist <= m3).astype(jnp.float32)
    acc = jax.lax.dot_general(
        maskf, caug_ref[0], (((1,), (0,)), ((), ())),
        preferred_element_type=jnp.float32,
        precision=jax.lax.Precision.HIGHEST,
    )
    out_ref[0] = acc[:, 0:3] / acc[:, 3:4]


def kernel(points1, points2, colors1):
    b, n2, _ = points2.shape
    n1 = points1.shape[1]
    p1t = jnp.transpose(points1, (0, 2, 1))
    ones = jnp.ones((b, n1, 1), dtype=colors1.dtype)
    caug = jnp.concatenate([colors1, ones], axis=-1)

    grid = (b, n2 // _R)
    out = pl.pallas_call(
        _knn_body,
        grid=grid,
        in_specs=[
            pl.BlockSpec((1, _R, 3), lambda i, j: (i, j, 0)),
            pl.BlockSpec((1, 3, n1), lambda i, j: (i, 0, 0)),
            pl.BlockSpec((1, n1, 4), lambda i, j: (i, 0, 0)),
        ],
        out_specs=pl.BlockSpec((1, _R, 3), lambda i, j: (i, j, 0)),
        out_shape=jax.ShapeDtypeStruct((b, n2, 3), jnp.float32),
    )(points2, p1t, caug)
    return out
